# baseline (speedup 1.0000x reference)
import dataclasses

import jax
import jax.numpy as jnp
from jax import lax
from jax.experimental import pallas as pl
from jax.experimental.pallas import tpu as pltpu
from jax.experimental.pallas import tpu_sc as plsc

N = 10000
E = 320000
D = 128
G = 256
NC = 2
NS = 16
NW = NC * NS
EPW = E // NW
K = 40
KP = 48
NBLK = EPW // K
KD = 2000
NBLKD = EPW // KD
NPAD = 10240
ROWS_PT = NPAD // NS

_MESH = plsc.VectorSubcoreMesh(core_axis_name="c", subcore_axis_name="s",
                               num_cores=NC, num_subcores=NS)

_HIGH = lax.Precision.HIGHEST

_SC_PARAMS = pltpu.CompilerParams()
if "needs_layout_passes" in pltpu.CompilerParams.__dataclass_fields__:
    _SC_PARAMS = dataclasses.replace(_SC_PARAMS, needs_layout_passes=False)


def _dot(a, b, dims):
    return lax.dot_general(a, b, (dims, ((), ())), precision=_HIGH,
                           preferred_element_type=jnp.float32)


def _dot_t(a, w):
    return _dot(a, w, ((1,), (1,)))


def _deg_body(row_h, zerosn_h, degp_h, hist_v, idx_v, sem):
    c = lax.axis_index("c")
    s = lax.axis_index("s")
    wid = c * NS + s
    pltpu.async_copy(zerosn_h, hist_v, sem).wait()
    base = wid * EPW
    ones = jnp.full((16,), 1.0, jnp.float32)

    @pl.loop(0, NBLKD)
    def _(b):
        pltpu.sync_copy(row_h.at[pl.ds(base + b * KD, KD)], idx_v)

        @pl.loop(0, KD // 16)
        def _(i):
            idx16 = idx_v[pl.ds(i * 16, 16)]
            plsc.addupdate_scatter(hist_v, [idx16], ones)

    pltpu.sync_copy(hist_v, degp_h.at[wid])


def _deg_kernel(row, zerosn):
    return pl.kernel(
        _deg_body,
        out_type=jax.ShapeDtypeStruct((NW, NPAD), jnp.float32),
        mesh=_MESH,
        scratch_types=[
            pltpu.VMEM((NPAD,), jnp.float32),
            pltpu.VMEM((KD,), jnp.int32),
            pltpu.SemaphoreType.DMA,
        ],
        compiler_params=_SC_PARAMS,
    )(row, zerosn)


def _msg_body(hw2_h, btab_h, packed_h, zeros_h, aggp_h,
              agg_sh, idx0, idx1, gath0, gath1, bond0, bond1, msg0, msg1,
              colb0, colb1,
              sem_g0, sem_g1, sem_b0, sem_b1, sem_i0, sem_i1, sem_s0, sem_s1):
    c = lax.axis_index("c")
    s = lax.axis_index("s")
    wid = c * NS + s
    pltpu.sync_copy(zeros_h, agg_sh.at[pl.ds(s * ROWS_PT, ROWS_PT)])
    plsc.subcore_barrier()
    base = wid * NBLK

    def compute(gath_v, bond_v, msg_v):
        @pl.loop(0, K // 4)
        def _(i):
            for kk in range(4):
                k = i * 4 + kk
                dv = gath_v[k, pl.ds(D, 16)]
                for j in range(8):
                    g = gath_v[k, pl.ds(16 * j, 16)]
                    bb = bond_v[k, pl.ds(16 * j, 16)]
                    msg_v[k, pl.ds(16 * j, 16)] = jnp.maximum(g + dv * bb,
                                                              0.0)

    def issue_gathers(idx_v, gath_v, bond_v, sg, sb):
        pltpu.async_copy(hw2_h.at[idx_v.at[0, pl.ds(0, K)]], gath_v, sg)
        pltpu.async_copy(btab_h.at[idx_v.at[2, pl.ds(0, K)]], bond_v, sb)

    def save_cols(idx_v, colb):
        for t in range(KP // 16):
            colb[pl.ds(16 * t, 16)] = idx_v[1, pl.ds(16 * t, 16)]

    def scatter(msg_v, colb, sem):
        pltpu.async_copy(msg_v, agg_sh.at[colb.at[pl.ds(0, K)]], sem,
                         add=True)

    def scatter_wait(msg_v, colb, sem):
        pltpu.make_async_copy(msg_v, agg_sh.at[colb.at[pl.ds(0, K)]],
                              sem).wait()

    pltpu.sync_copy(packed_h.at[base], idx0)
    issue_gathers(idx0, gath0, bond0, sem_g0, sem_b0)
    pltpu.async_copy(packed_h.at[base + 1], idx1, sem_i1)

    @pl.loop(0, NBLK // 2)
    def _(i):
        gb = base + 2 * i
        pltpu.make_async_copy(hw2_h.at[idx0.at[0, pl.ds(0, K)]], gath0,
                              sem_g0).wait()
        pltpu.make_async_copy(btab_h.at[idx0.at[2, pl.ds(0, K)]], bond0,
                              sem_b0).wait()
        pltpu.make_async_copy(packed_h.at[gb + 1], idx1, sem_i1).wait()
        issue_gathers(idx1, gath1, bond1, sem_g1, sem_b1)

        @pl.when(i > 0)
        def _():
            scatter_wait(msg0, colb0, sem_s0)

        compute(gath0, bond0, msg0)
        save_cols(idx0, colb0)
        scatter(msg0, colb0, sem_s0)

        @pl.when(i < NBLK // 2 - 1)
        def _():
            pltpu.async_copy(packed_h.at[gb + 2], idx0, sem_i0)

        pltpu.make_async_copy(hw2_h.at[idx1.at[0, pl.ds(0, K)]], gath1,
                              sem_g1).wait()
        pltpu.make_async_copy(btab_h.at[idx1.at[2, pl.ds(0, K)]], bond1,
                              sem_b1).wait()

        @pl.when(i < NBLK // 2 - 1)
        def _():
            pltpu.make_async_copy(packed_h.at[gb + 2], idx0, sem_i0).wait()
            issue_gathers(idx0, gath0, bond0, sem_g0, sem_b0)

        @pl.when(i > 0)
        def _():
            scatter_wait(msg1, colb1, sem_s1)

        compute(gath1, bond1, msg1)
        save_cols(idx1, colb1)
        scatter(msg1, colb1, sem_s1)

        @pl.when(i < NBLK // 2 - 1)
        def _():
            pltpu.async_copy(packed_h.at[gb + 3], idx1, sem_i1)

    scatter_wait(msg0, colb0, sem_s0)
    scatter_wait(msg1, colb1, sem_s1)
    plsc.subcore_barrier()
    pltpu.sync_copy(agg_sh.at[pl.ds(s * ROWS_PT, ROWS_PT)],
                    aggp_h.at[c, pl.ds(s * ROWS_PT, ROWS_PT)])


def _msg_kernel(hw2cat, btab, packed, zeros):
    return pl.kernel(
        _msg_body,
        out_type=jax.ShapeDtypeStruct((NC, NPAD, D), jnp.float32),
        mesh=_MESH,
        scratch_types=[
            pltpu.VMEM_SHARED((NPAD, D), jnp.float32),
            pltpu.VMEM((3, KP), jnp.int32),
            pltpu.VMEM((3, KP), jnp.int32),
            pltpu.VMEM((K, 2 * D), jnp.float32),
            pltpu.VMEM((K, 2 * D), jnp.float32),
            pltpu.VMEM((K, D), jnp.float32),
            pltpu.VMEM((K, D), jnp.float32),
            pltpu.VMEM((K, D), jnp.float32),
            pltpu.VMEM((K, D), jnp.float32),
            pltpu.VMEM((KP,), jnp.int32),
            pltpu.VMEM((KP,), jnp.int32),
            pltpu.SemaphoreType.DMA,
            pltpu.SemaphoreType.DMA,
            pltpu.SemaphoreType.DMA,
            pltpu.SemaphoreType.DMA,
            pltpu.SemaphoreType.DMA,
            pltpu.SemaphoreType.DMA,
            pltpu.SemaphoreType.DMA,
            pltpu.SemaphoreType.DMA,
        ],
        compiler_params=_SC_PARAMS,
    )(hw2cat, btab, packed, zeros)


_NB = 1000
_NBLKS = N // _NB


def _tc_deg_body(degp_ref, discol_ref, deginv_ref):
    deg = _dot(degp_ref[:, :N], jnp.ones((NW, 1), jnp.float32),
               ((0,), (0,))) + 1.0
    discol_ref[...] = jnp.broadcast_to(lax.rsqrt(deg), (N, 16))
    deginv_ref[...] = jnp.broadcast_to(1.0 / deg, (N, 16))


def _tc_deg(degp):
    return pl.pallas_call(
        _tc_deg_body,
        out_shape=[
            jax.ShapeDtypeStruct((N, 16), jnp.float32),
            jax.ShapeDtypeStruct((N, 16), jnp.float32),
        ],
    )(degp)


def _tc_prep_body(x_ref, at_ref, discol_ref, deginv_ref, w0_ref, root0_ref,
                  hw2_ref, self0_ref):
    h0 = jnp.zeros((_NB, D), jnp.float32)
    for f in range(9):
        oh = (x_ref[:, f:f + 1]
              == lax.broadcasted_iota(jnp.int32, (1, 120), 1)).astype(jnp.float32)
        h0 = h0 + _dot(oh, at_ref[f], ((1,), (0,)))
    dis = discol_ref[:, 0:1]
    dinv = deginv_ref[:, 0:1]
    hw0 = _dot_t(h0, w0_ref[...])
    hw2_ref[:, :D] = hw0 * dis
    hw2_ref[:, D:] = jnp.broadcast_to(dis, (_NB, D))
    self0_ref[...] = jnp.maximum(hw0 + root0_ref[...], 0.0) * dinv


def _tc_prep(x, atom_table, discol, deginv, W0, root0):
    return pl.pallas_call(
        _tc_prep_body,
        grid=(_NBLKS,),
        in_specs=[
            pl.BlockSpec((_NB, 9), lambda i: (i, 0)),
            pl.BlockSpec((9, 120, D), lambda i: (0, 0, 0)),
            pl.BlockSpec((_NB, 16), lambda i: (i, 0)),
            pl.BlockSpec((_NB, 16), lambda i: (i, 0)),
            pl.BlockSpec((D, D), lambda i: (0, 0)),
            pl.BlockSpec((1, D), lambda i: (0, 0)),
        ],
        out_specs=[
            pl.BlockSpec((_NB, 2 * D), lambda i: (i, 0)),
            pl.BlockSpec((_NB, D), lambda i: (i, 0)),
        ],
        out_shape=[
            jax.ShapeDtypeStruct((N, 2 * D), jnp.float32),
            jax.ShapeDtypeStruct((N, D), jnp.float32),
        ],
    )(x, atom_table, discol, deginv, W0, root0)


def _h_block(aggp_ref, discol_ref, self_ref):
    return ((aggp_ref[0] + aggp_ref[1]) * discol_ref[:, 0:1] + self_ref[...])


def _tc_stats_body(aggp_ref, discol_ref, self_ref, sums_ref):
    i = pl.program_id(0)

    @pl.when(i == 0)
    def _():
        sums_ref[...] = jnp.zeros((2, D), jnp.float32)

    h = _h_block(aggp_ref, discol_ref, self_ref)
    sums_ref[0:1, :] += jnp.sum(h, axis=0, keepdims=True)
    sums_ref[1:2, :] += jnp.sum(h * h, axis=0, keepdims=True)


def _tc_stats(aggp, discol, selfh):
    return pl.pallas_call(
        _tc_stats_body,
        grid=(_NBLKS,),
        in_specs=[
            pl.BlockSpec((NC, _NB, D), lambda i: (0, i, 0)),
            pl.BlockSpec((_NB, 16), lambda i: (i, 0)),
            pl.BlockSpec((_NB, D), lambda i: (i, 0)),
        ],
        out_specs=pl.BlockSpec((2, D), lambda i: (0, 0)),
        out_shape=jax.ShapeDtypeStruct((2, D), jnp.float32),
    )(aggp, discol, selfh)


def _tc_apply_body(aggp_ref, discol_ref, deginv_ref, self_ref, sums_ref,
                   g_ref, b_ref, wn_ref, rootn_ref, hw2_ref, selfn_ref):
    mean = sums_ref[0:1, :] / N
    var = sums_ref[1:2, :] / N - mean * mean
    scale = lax.rsqrt(var + 1e-5) * g_ref[...]
    shift = b_ref[...] - mean * scale
    dis = discol_ref[:, 0:1]
    dinv = deginv_ref[:, 0:1]
    h = jnp.maximum(_h_block(aggp_ref, discol_ref, self_ref) * scale + shift,
                    0.0)
    hwn = _dot_t(h, wn_ref[...])
    hw2_ref[:, :D] = hwn * dis
    hw2_ref[:, D:] = jnp.broadcast_to(dis, (_NB, D))
    selfn_ref[...] = jnp.maximum(hwn + rootn_ref[...], 0.0) * dinv


def _tc_combine(aggp, discol, deginv, selfh, bn_g, bn_b, Wn, rootn):
    sums = _tc_stats(aggp, discol, selfh)
    return pl.pallas_call(
        _tc_apply_body,
        grid=(_NBLKS,),
        in_specs=[
            pl.BlockSpec((NC, _NB, D), lambda i: (0, i, 0)),
            pl.BlockSpec((_NB, 16), lambda i: (i, 0)),
            pl.BlockSpec((_NB, 16), lambda i: (i, 0)),
            pl.BlockSpec((_NB, D), lambda i: (i, 0)),
            pl.BlockSpec((2, D), lambda i: (0, 0)),
            pl.BlockSpec((1, D), lambda i: (0, 0)),
            pl.BlockSpec((1, D), lambda i: (0, 0)),
            pl.BlockSpec((D, D), lambda i: (0, 0)),
            pl.BlockSpec((1, D), lambda i: (0, 0)),
        ],
        out_specs=[
            pl.BlockSpec((_NB, 2 * D), lambda i: (i, 0)),
            pl.BlockSpec((_NB, D), lambda i: (i, 0)),
        ],
        out_shape=[
            jax.ShapeDtypeStruct((N, 2 * D), jnp.float32),
            jax.ShapeDtypeStruct((N, D), jnp.float32),
        ],
    )(aggp, discol, deginv, selfh, sums, bn_g, bn_b, Wn, rootn)


def _tc_final_body(aggp_ref, discol_ref, self_ref, batch_ref, wp_ref, out_ref,
                   pooled_sc, counts_sc):
    i = pl.program_id(0)

    @pl.when(i == 0)
    def _():
        pooled_sc[...] = jnp.zeros((G, D), jnp.float32)
        counts_sc[...] = jnp.zeros((G, 1), jnp.float32)

    h = _h_block(aggp_ref, discol_ref, self_ref)
    oh = (batch_ref[...]
          == lax.broadcasted_iota(jnp.int32, (1, G), 1)).astype(jnp.float32)
    pooled_sc[...] += _dot(oh, h, ((0,), (0,)))
    counts_sc[...] += _dot(oh, jnp.ones((_NB, 1), jnp.float32), ((0,), (0,)))

    @pl.when(i == _NBLKS - 1)
    def _():
        pooled = pooled_sc[...] / jnp.maximum(counts_sc[:, 0:1], 1.0)
        out_ref[...] = _dot_t(pooled, wp_ref[...])


def _tc_final(aggp, discol, selfh, batch2, Wp):
    return pl.pallas_call(
        _tc_final_body,
        grid=(_NBLKS,),
        in_specs=[
            pl.BlockSpec((NC, _NB, D), lambda i: (0, i, 0)),
            pl.BlockSpec((_NB, 16), lambda i: (i, 0)),
            pl.BlockSpec((_NB, D), lambda i: (i, 0)),
            pl.BlockSpec((_NB, 1), lambda i: (i, 0)),
            pl.BlockSpec((D, D), lambda i: (0, 0)),
        ],
        out_specs=pl.BlockSpec((G, D), lambda i: (0, 0)),
        out_shape=jax.ShapeDtypeStruct((G, Wp.shape[0]), jnp.float32),
        scratch_shapes=[
            pltpu.VMEM((G, D), jnp.float32),
            pltpu.VMEM((G, 1), jnp.float32),
        ],
    )(aggp, discol, selfh, batch2, Wp)


def _btab(bond_table):
    t = (bond_table[0][:, None, None, :] + bond_table[1][None, :, None, :]
         + bond_table[2][None, None, :, :]).reshape(125, D)
    return jnp.pad(t, ((0, 3), (0, 0)))


def kernel(x, edge_index, edge_attr, batch, atom_table, W0, root0, bond0,
           W1, root1, bond1, W2, root2, bond2, bn0_g, bn0_b, bn1_g, bn1_b, Wp):
    row = edge_index[0].astype(jnp.int32)
    col = edge_index[1].astype(jnp.int32)
    combo = (edge_attr[:, 0] * 25 + edge_attr[:, 1] * 5
             + edge_attr[:, 2]).astype(jnp.int32)
    packed = jnp.stack(
        [row.reshape(NW, NBLK, K), col.reshape(NW, NBLK, K),
         combo.reshape(NW, NBLK, K)], axis=2).reshape(NW * NBLK, 3, K)
    packed = jnp.pad(packed, ((0, 0), (0, 0), (0, KP - K)))
    zeros = jnp.zeros((ROWS_PT, D), jnp.float32)
    zerosn = jnp.zeros((NPAD,), jnp.float32)

    degp = _deg_kernel(row, zerosn)
    discol, deginv = _tc_deg(degp)
    hw2, selfh = _tc_prep(x.astype(jnp.int32), atom_table, discol, deginv,
                          W0, root0)

    aggp = _msg_kernel(hw2, _btab(bond0), packed, zeros)
    hw2, selfh = _tc_combine(aggp, discol, deginv, selfh, bn0_g.reshape(1, D),
                             bn0_b.reshape(1, D), W1, root1)

    aggp = _msg_kernel(hw2, _btab(bond1), packed, zeros)
    hw2, selfh = _tc_combine(aggp, discol, deginv, selfh, bn1_g.reshape(1, D),
                             bn1_b.reshape(1, D), W2, root2)

    aggp = _msg_kernel(hw2, _btab(bond2), packed, zeros)
    return _tc_final(aggp, discol, selfh, batch.astype(jnp.int32).reshape(N, 1),
                     Wp)


# speedup vs baseline: 1.5356x; 1.5356x over previous
import dataclasses

import jax
import jax.numpy as jnp
from jax import lax
from jax.experimental import pallas as pl
from jax.experimental.pallas import tpu as pltpu
from jax.experimental.pallas import tpu_sc as plsc

N = 10000
E = 320000
D = 128
G = 256
NC = 2
NS = 16
NW = NC * NS
EPW = E // NW
K = 40
KP = 48
NBLK = EPW // K
KD = 2000
NBLKD = EPW // KD
NPAD = 10240
ROWS_PT = NPAD // NS

_MESH = plsc.VectorSubcoreMesh(core_axis_name="c", subcore_axis_name="s",
                               num_cores=NC, num_subcores=NS)

_HIGH = lax.Precision.HIGHEST

_SC_PARAMS = pltpu.CompilerParams()
if "needs_layout_passes" in pltpu.CompilerParams.__dataclass_fields__:
    _SC_PARAMS = dataclasses.replace(_SC_PARAMS, needs_layout_passes=False)


def _dot(a, b, dims):
    return lax.dot_general(a, b, (dims, ((), ())), precision=_HIGH,
                           preferred_element_type=jnp.float32)


def _dot_t(a, w):
    return _dot(a, w, ((1,), (1,)))


def _deg_body(row_h, zerosn_h, degp_h, hist_v, idx_v, sem):
    c = lax.axis_index("c")
    s = lax.axis_index("s")
    wid = c * NS + s
    pltpu.async_copy(zerosn_h, hist_v, sem).wait()
    base = wid * EPW
    ones = jnp.full((16,), 1.0, jnp.float32)

    @pl.loop(0, NBLKD)
    def _(b):
        pltpu.sync_copy(row_h.at[pl.ds(base + b * KD, KD)], idx_v)

        @pl.loop(0, KD // 16)
        def _(i):
            idx16 = idx_v[pl.ds(i * 16, 16)]
            plsc.addupdate_scatter(hist_v, [idx16], ones)

    pltpu.sync_copy(hist_v, degp_h.at[wid])


def _deg_kernel(row, zerosn):
    return pl.kernel(
        _deg_body,
        out_type=jax.ShapeDtypeStruct((NW, NPAD), jnp.float32),
        mesh=_MESH,
        scratch_types=[
            pltpu.VMEM((NPAD,), jnp.float32),
            pltpu.VMEM((KD,), jnp.int32),
            pltpu.SemaphoreType.DMA,
        ],
        compiler_params=_SC_PARAMS,
    )(row, zerosn)


def _msg_body(hw2_h, btab_h, packed_h, zeros_h, aggp_h,
              agg_sh, idx0, idx1, gath0, gath1, bond0, bond1, msg0, msg1,
              colb0, colb1,
              sem_g0, sem_g1, sem_b0, sem_b1, sem_i0, sem_i1, sem_s0, sem_s1):
    c = lax.axis_index("c")
    s = lax.axis_index("s")
    wid = c * NS + s
    pltpu.sync_copy(zeros_h, agg_sh.at[pl.ds(s * ROWS_PT, ROWS_PT)])
    plsc.subcore_barrier()
    base = wid * NBLK

    def compute(gath_v, bond_v, msg_v):
        @plsc.parallel_loop(0, K, 1, unroll=4)
        def _(k):
            dv = gath_v[k, pl.ds(D, 16)]
            for j in range(8):
                g = gath_v[k, pl.ds(16 * j, 16)]
                bb = bond_v[k, pl.ds(16 * j, 16)]
                msg_v[k, pl.ds(16 * j, 16)] = jnp.maximum(g + dv * bb, 0.0)

    def issue_gathers(idx_v, gath_v, bond_v, sg, sb):
        pltpu.async_copy(hw2_h.at[idx_v.at[0, pl.ds(0, K)]], gath_v, sg)
        pltpu.async_copy(btab_h.at[idx_v.at[2, pl.ds(0, K)]], bond_v, sb)

    def save_cols(idx_v, colb):
        for t in range(KP // 16):
            colb[pl.ds(16 * t, 16)] = idx_v[1, pl.ds(16 * t, 16)]

    def scatter(msg_v, colb, sem):
        pltpu.async_copy(msg_v, agg_sh.at[colb.at[pl.ds(0, K)]], sem,
                         add=True)

    def scatter_wait(msg_v, colb, sem):
        pltpu.make_async_copy(msg_v, agg_sh.at[colb.at[pl.ds(0, K)]],
                              sem).wait()

    pltpu.sync_copy(packed_h.at[base], idx0)
    issue_gathers(idx0, gath0, bond0, sem_g0, sem_b0)
    pltpu.async_copy(packed_h.at[base + 1], idx1, sem_i1)

    @pl.loop(0, NBLK // 2)
    def _(i):
        gb = base + 2 * i
        pltpu.make_async_copy(hw2_h.at[idx0.at[0, pl.ds(0, K)]], gath0,
                              sem_g0).wait()
        pltpu.make_async_copy(btab_h.at[idx0.at[2, pl.ds(0, K)]], bond0,
                              sem_b0).wait()
        pltpu.make_async_copy(packed_h.at[gb + 1], idx1, sem_i1).wait()
        issue_gathers(idx1, gath1, bond1, sem_g1, sem_b1)

        @pl.when(i > 0)
        def _():
            scatter_wait(msg0, colb0, sem_s0)

        compute(gath0, bond0, msg0)
        save_cols(idx0, colb0)
        scatter(msg0, colb0, sem_s0)

        @pl.when(i < NBLK // 2 - 1)
        def _():
            pltpu.async_copy(packed_h.at[gb + 2], idx0, sem_i0)

        pltpu.make_async_copy(hw2_h.at[idx1.at[0, pl.ds(0, K)]], gath1,
                              sem_g1).wait()
        pltpu.make_async_copy(btab_h.at[idx1.at[2, pl.ds(0, K)]], bond1,
                              sem_b1).wait()

        @pl.when(i < NBLK // 2 - 1)
        def _():
            pltpu.make_async_copy(packed_h.at[gb + 2], idx0, sem_i0).wait()
            issue_gathers(idx0, gath0, bond0, sem_g0, sem_b0)

        @pl.when(i > 0)
        def _():
            scatter_wait(msg1, colb1, sem_s1)

        compute(gath1, bond1, msg1)
        save_cols(idx1, colb1)
        scatter(msg1, colb1, sem_s1)

        @pl.when(i < NBLK // 2 - 1)
        def _():
            pltpu.async_copy(packed_h.at[gb + 3], idx1, sem_i1)

    scatter_wait(msg0, colb0, sem_s0)
    scatter_wait(msg1, colb1, sem_s1)
    plsc.subcore_barrier()
    pltpu.sync_copy(agg_sh.at[pl.ds(s * ROWS_PT, ROWS_PT)],
                    aggp_h.at[c, pl.ds(s * ROWS_PT, ROWS_PT)])


def _msg_kernel(hw2cat, btab, packed, zeros):
    return pl.kernel(
        _msg_body,
        out_type=jax.ShapeDtypeStruct((NC, NPAD, D), jnp.float32),
        mesh=_MESH,
        scratch_types=[
            pltpu.VMEM_SHARED((NPAD, D), jnp.float32),
            pltpu.VMEM((3, KP), jnp.int32),
            pltpu.VMEM((3, KP), jnp.int32),
            pltpu.VMEM((K, 2 * D), jnp.float32),
            pltpu.VMEM((K, 2 * D), jnp.float32),
            pltpu.VMEM((K, D), jnp.float32),
            pltpu.VMEM((K, D), jnp.float32),
            pltpu.VMEM((K, D), jnp.float32),
            pltpu.VMEM((K, D), jnp.float32),
            pltpu.VMEM((KP,), jnp.int32),
            pltpu.VMEM((KP,), jnp.int32),
            pltpu.SemaphoreType.DMA,
            pltpu.SemaphoreType.DMA,
            pltpu.SemaphoreType.DMA,
            pltpu.SemaphoreType.DMA,
            pltpu.SemaphoreType.DMA,
            pltpu.SemaphoreType.DMA,
            pltpu.SemaphoreType.DMA,
            pltpu.SemaphoreType.DMA,
        ],
        compiler_params=_SC_PARAMS,
    )(hw2cat, btab, packed, zeros)


_NB = 1000
_NBLKS = N // _NB


def _tc_deg_body(degp_ref, discol_ref, deginv_ref):
    deg = _dot(degp_ref[:, :N], jnp.ones((NW, 1), jnp.float32),
               ((0,), (0,))) + 1.0
    discol_ref[...] = jnp.broadcast_to(lax.rsqrt(deg), (N, 16))
    deginv_ref[...] = jnp.broadcast_to(1.0 / deg, (N, 16))


def _tc_deg(degp):
    return pl.pallas_call(
        _tc_deg_body,
        out_shape=[
            jax.ShapeDtypeStruct((N, 16), jnp.float32),
            jax.ShapeDtypeStruct((N, 16), jnp.float32),
        ],
    )(degp)


def _tc_prep_body(x_ref, at_ref, discol_ref, deginv_ref, w0_ref, root0_ref,
                  hw2_ref, self0_ref):
    h0 = jnp.zeros((_NB, D), jnp.float32)
    for f in range(9):
        oh = (x_ref[:, f:f + 1]
              == lax.broadcasted_iota(jnp.int32, (1, 120), 1)).astype(jnp.float32)
        h0 = h0 + _dot(oh, at_ref[f], ((1,), (0,)))
    dis = discol_ref[:, 0:1]
    dinv = deginv_ref[:, 0:1]
    hw0 = _dot_t(h0, w0_ref[...])
    hw2_ref[:, :D] = hw0 * dis
    hw2_ref[:, D:] = jnp.broadcast_to(dis, (_NB, D))
    self0_ref[...] = jnp.maximum(hw0 + root0_ref[...], 0.0) * dinv


def _tc_prep(x, atom_table, discol, deginv, W0, root0):
    return pl.pallas_call(
        _tc_prep_body,
        grid=(_NBLKS,),
        in_specs=[
            pl.BlockSpec((_NB, 9), lambda i: (i, 0)),
            pl.BlockSpec((9, 120, D), lambda i: (0, 0, 0)),
            pl.BlockSpec((_NB, 16), lambda i: (i, 0)),
            pl.BlockSpec((_NB, 16), lambda i: (i, 0)),
            pl.BlockSpec((D, D), lambda i: (0, 0)),
            pl.BlockSpec((1, D), lambda i: (0, 0)),
        ],
        out_specs=[
            pl.BlockSpec((_NB, 2 * D), lambda i: (i, 0)),
            pl.BlockSpec((_NB, D), lambda i: (i, 0)),
        ],
        out_shape=[
            jax.ShapeDtypeStruct((N, 2 * D), jnp.float32),
            jax.ShapeDtypeStruct((N, D), jnp.float32),
        ],
    )(x, atom_table, discol, deginv, W0, root0)


def _h_block(aggp_ref, discol_ref, self_ref):
    return ((aggp_ref[0] + aggp_ref[1]) * discol_ref[:, 0:1] + self_ref[...])


def _tc_stats_body(aggp_ref, discol_ref, self_ref, sums_ref):
    i = pl.program_id(0)

    @pl.when(i == 0)
    def _():
        sums_ref[...] = jnp.zeros((2, D), jnp.float32)

    h = _h_block(aggp_ref, discol_ref, self_ref)
    sums_ref[0:1, :] += jnp.sum(h, axis=0, keepdims=True)
    sums_ref[1:2, :] += jnp.sum(h * h, axis=0, keepdims=True)


def _tc_stats(aggp, discol, selfh):
    return pl.pallas_call(
        _tc_stats_body,
        grid=(_NBLKS,),
        in_specs=[
            pl.BlockSpec((NC, _NB, D), lambda i: (0, i, 0)),
            pl.BlockSpec((_NB, 16), lambda i: (i, 0)),
            pl.BlockSpec((_NB, D), lambda i: (i, 0)),
        ],
        out_specs=pl.BlockSpec((2, D), lambda i: (0, 0)),
        out_shape=jax.ShapeDtypeStruct((2, D), jnp.float32),
    )(aggp, discol, selfh)


def _tc_apply_body(aggp_ref, discol_ref, deginv_ref, self_ref, sums_ref,
                   g_ref, b_ref, wn_ref, rootn_ref, hw2_ref, selfn_ref):
    mean = sums_ref[0:1, :] / N
    var = sums_ref[1:2, :] / N - mean * mean
    scale = lax.rsqrt(var + 1e-5) * g_ref[...]
    shift = b_ref[...] - mean * scale
    dis = discol_ref[:, 0:1]
    dinv = deginv_ref[:, 0:1]
    h = jnp.maximum(_h_block(aggp_ref, discol_ref, self_ref) * scale + shift,
                    0.0)
    hwn = _dot_t(h, wn_ref[...])
    hw2_ref[:, :D] = hwn * dis
    hw2_ref[:, D:] = jnp.broadcast_to(dis, (_NB, D))
    selfn_ref[...] = jnp.maximum(hwn + rootn_ref[...], 0.0) * dinv


def _tc_combine(aggp, discol, deginv, selfh, bn_g, bn_b, Wn, rootn):
    sums = _tc_stats(aggp, discol, selfh)
    return pl.pallas_call(
        _tc_apply_body,
        grid=(_NBLKS,),
        in_specs=[
            pl.BlockSpec((NC, _NB, D), lambda i: (0, i, 0)),
            pl.BlockSpec((_NB, 16), lambda i: (i, 0)),
            pl.BlockSpec((_NB, 16), lambda i: (i, 0)),
            pl.BlockSpec((_NB, D), lambda i: (i, 0)),
            pl.BlockSpec((2, D), lambda i: (0, 0)),
            pl.BlockSpec((1, D), lambda i: (0, 0)),
            pl.BlockSpec((1, D), lambda i: (0, 0)),
            pl.BlockSpec((D, D), lambda i: (0, 0)),
            pl.BlockSpec((1, D), lambda i: (0, 0)),
        ],
        out_specs=[
            pl.BlockSpec((_NB, 2 * D), lambda i: (i, 0)),
            pl.BlockSpec((_NB, D), lambda i: (i, 0)),
        ],
        out_shape=[
            jax.ShapeDtypeStruct((N, 2 * D), jnp.float32),
            jax.ShapeDtypeStruct((N, D), jnp.float32),
        ],
    )(aggp, discol, deginv, selfh, sums, bn_g, bn_b, Wn, rootn)


def _tc_final_body(aggp_ref, discol_ref, self_ref, batch_ref, wp_ref, out_ref,
                   pooled_sc, counts_sc):
    i = pl.program_id(0)

    @pl.when(i == 0)
    def _():
        pooled_sc[...] = jnp.zeros((G, D), jnp.float32)
        counts_sc[...] = jnp.zeros((G, 1), jnp.float32)

    h = _h_block(aggp_ref, discol_ref, self_ref)
    oh = (batch_ref[...]
          == lax.broadcasted_iota(jnp.int32, (1, G), 1)).astype(jnp.float32)
    pooled_sc[...] += _dot(oh, h, ((0,), (0,)))
    counts_sc[...] += _dot(oh, jnp.ones((_NB, 1), jnp.float32), ((0,), (0,)))

    @pl.when(i == _NBLKS - 1)
    def _():
        pooled = pooled_sc[...] / jnp.maximum(counts_sc[:, 0:1], 1.0)
        out_ref[...] = _dot_t(pooled, wp_ref[...])


def _tc_final(aggp, discol, selfh, batch2, Wp):
    return pl.pallas_call(
        _tc_final_body,
        grid=(_NBLKS,),
        in_specs=[
            pl.BlockSpec((NC, _NB, D), lambda i: (0, i, 0)),
            pl.BlockSpec((_NB, 16), lambda i: (i, 0)),
            pl.BlockSpec((_NB, D), lambda i: (i, 0)),
            pl.BlockSpec((_NB, 1), lambda i: (i, 0)),
            pl.BlockSpec((D, D), lambda i: (0, 0)),
        ],
        out_specs=pl.BlockSpec((G, D), lambda i: (0, 0)),
        out_shape=jax.ShapeDtypeStruct((G, Wp.shape[0]), jnp.float32),
        scratch_shapes=[
            pltpu.VMEM((G, D), jnp.float32),
            pltpu.VMEM((G, 1), jnp.float32),
        ],
    )(aggp, discol, selfh, batch2, Wp)


def _btab(bond_table):
    t = (bond_table[0][:, None, None, :] + bond_table[1][None, :, None, :]
         + bond_table[2][None, None, :, :]).reshape(125, D)
    return jnp.pad(t, ((0, 3), (0, 0)))


def kernel(x, edge_index, edge_attr, batch, atom_table, W0, root0, bond0,
           W1, root1, bond1, W2, root2, bond2, bn0_g, bn0_b, bn1_g, bn1_b, Wp):
    row = edge_index[0].astype(jnp.int32)
    col = edge_index[1].astype(jnp.int32)
    combo = (edge_attr[:, 0] * 25 + edge_attr[:, 1] * 5
             + edge_attr[:, 2]).astype(jnp.int32)
    packed = jnp.stack(
        [row.reshape(NW, NBLK, K), col.reshape(NW, NBLK, K),
         combo.reshape(NW, NBLK, K)], axis=2).reshape(NW * NBLK, 3, K)
    packed = jnp.pad(packed, ((0, 0), (0, 0), (0, KP - K)))
    zeros = jnp.zeros((ROWS_PT, D), jnp.float32)
    zerosn = jnp.zeros((NPAD,), jnp.float32)

    degp = _deg_kernel(row, zerosn)
    discol, deginv = _tc_deg(degp)
    hw2, selfh = _tc_prep(x.astype(jnp.int32), atom_table, discol, deginv,
                          W0, root0)

    aggp = _msg_kernel(hw2, _btab(bond0), packed, zeros)
    hw2, selfh = _tc_combine(aggp, discol, deginv, selfh, bn0_g.reshape(1, D),
                             bn0_b.reshape(1, D), W1, root1)

    aggp = _msg_kernel(hw2, _btab(bond1), packed, zeros)
    hw2, selfh = _tc_combine(aggp, discol, deginv, selfh, bn1_g.reshape(1, D),
                             bn1_b.reshape(1, D), W2, root2)

    aggp = _msg_kernel(hw2, _btab(bond2), packed, zeros)
    return _tc_final(aggp, discol, selfh, batch.astype(jnp.int32).reshape(N, 1),
                     Wp)


# speedup vs baseline: 1.5361x; 1.0003x over previous
import dataclasses

import jax
import jax.numpy as jnp
from jax import lax
from jax.experimental import pallas as pl
from jax.experimental.pallas import tpu as pltpu
from jax.experimental.pallas import tpu_sc as plsc

N = 10000
E = 320000
D = 128
G = 256
NC = 2
NS = 16
NW = NC * NS
EPW = E // NW
K = 40
KP = 48
NBLK = EPW // K
KD = 2000
NBLKD = EPW // KD
NPAD = 10240
ROWS_PT = NPAD // NS

_MESH = plsc.VectorSubcoreMesh(core_axis_name="c", subcore_axis_name="s",
                               num_cores=NC, num_subcores=NS)

_HIGH = lax.Precision.HIGHEST

_SC_PARAMS = pltpu.CompilerParams()
if "needs_layout_passes" in pltpu.CompilerParams.__dataclass_fields__:
    _SC_PARAMS = dataclasses.replace(_SC_PARAMS, needs_layout_passes=False)


def _dot(a, b, dims):
    return lax.dot_general(a, b, (dims, ((), ())), precision=_HIGH,
                           preferred_element_type=jnp.float32)


def _dot_t(a, w):
    return _dot(a, w, ((1,), (1,)))


def _deg_body(row_h, zerosn_h, degp_h, hist_v, idx_v, sem):
    c = lax.axis_index("c")
    s = lax.axis_index("s")
    wid = c * NS + s
    pltpu.async_copy(zerosn_h, hist_v, sem).wait()
    base = wid * EPW
    ones = jnp.full((16,), 1.0, jnp.float32)

    @pl.loop(0, NBLKD)
    def _(b):
        pltpu.sync_copy(row_h.at[pl.ds(base + b * KD, KD)], idx_v)

        @pl.loop(0, KD // 16)
        def _(i):
            idx16 = idx_v[pl.ds(i * 16, 16)]
            plsc.addupdate_scatter(hist_v, [idx16], ones)

    pltpu.sync_copy(hist_v, degp_h.at[wid])


def _deg_kernel(row, zerosn):
    return pl.kernel(
        _deg_body,
        out_type=jax.ShapeDtypeStruct((NW, NPAD), jnp.float32),
        mesh=_MESH,
        scratch_types=[
            pltpu.VMEM((NPAD,), jnp.float32),
            pltpu.VMEM((KD,), jnp.int32),
            pltpu.SemaphoreType.DMA,
        ],
        compiler_params=_SC_PARAMS,
    )(row, zerosn)


def _msg_body(hw2_h, btab_h, packed_h, zeros_h, aggp_h,
              agg_sh, idx0, idx1, gath0, gath1, bond0, bond1, msg0, msg1,
              colb0, colb1,
              sem_g0, sem_g1, sem_b0, sem_b1, sem_i0, sem_i1, sem_s0, sem_s1):
    c = lax.axis_index("c")
    s = lax.axis_index("s")
    wid = c * NS + s
    pltpu.sync_copy(zeros_h, agg_sh.at[pl.ds(s * ROWS_PT, ROWS_PT)])
    plsc.subcore_barrier()
    base = wid * NBLK

    def compute(gath_v, bond_v, msg_v):
        @plsc.parallel_loop(0, K, 1, unroll=8)
        def _(k):
            dv = gath_v[k, pl.ds(D, 16)]
            for j in range(8):
                g = gath_v[k, pl.ds(16 * j, 16)]
                bb = bond_v[k, pl.ds(16 * j, 16)]
                msg_v[k, pl.ds(16 * j, 16)] = jnp.maximum(g + dv * bb, 0.0)

    def issue_gathers(idx_v, gath_v, bond_v, sg, sb):
        pltpu.async_copy(hw2_h.at[idx_v.at[0, pl.ds(0, K)]], gath_v, sg)
        pltpu.async_copy(btab_h.at[idx_v.at[2, pl.ds(0, K)]], bond_v, sb)

    def save_cols(idx_v, colb):
        for t in range(KP // 16):
            colb[pl.ds(16 * t, 16)] = idx_v[1, pl.ds(16 * t, 16)]

    def scatter(msg_v, colb, sem):
        pltpu.async_copy(msg_v, agg_sh.at[colb.at[pl.ds(0, K)]], sem,
                         add=True)

    def scatter_wait(msg_v, colb, sem):
        pltpu.make_async_copy(msg_v, agg_sh.at[colb.at[pl.ds(0, K)]],
                              sem).wait()

    pltpu.sync_copy(packed_h.at[base], idx0)
    issue_gathers(idx0, gath0, bond0, sem_g0, sem_b0)
    pltpu.async_copy(packed_h.at[base + 1], idx1, sem_i1)

    @pl.loop(0, NBLK // 2)
    def _(i):
        gb = base + 2 * i
        pltpu.make_async_copy(hw2_h.at[idx0.at[0, pl.ds(0, K)]], gath0,
                              sem_g0).wait()
        pltpu.make_async_copy(btab_h.at[idx0.at[2, pl.ds(0, K)]], bond0,
                              sem_b0).wait()
        pltpu.make_async_copy(packed_h.at[gb + 1], idx1, sem_i1).wait()
        issue_gathers(idx1, gath1, bond1, sem_g1, sem_b1)

        @pl.when(i > 0)
        def _():
            scatter_wait(msg0, colb0, sem_s0)

        compute(gath0, bond0, msg0)
        save_cols(idx0, colb0)
        scatter(msg0, colb0, sem_s0)

        @pl.when(i < NBLK // 2 - 1)
        def _():
            pltpu.async_copy(packed_h.at[gb + 2], idx0, sem_i0)

        pltpu.make_async_copy(hw2_h.at[idx1.at[0, pl.ds(0, K)]], gath1,
                              sem_g1).wait()
        pltpu.make_async_copy(btab_h.at[idx1.at[2, pl.ds(0, K)]], bond1,
                              sem_b1).wait()

        @pl.when(i < NBLK // 2 - 1)
        def _():
            pltpu.make_async_copy(packed_h.at[gb + 2], idx0, sem_i0).wait()
            issue_gathers(idx0, gath0, bond0, sem_g0, sem_b0)

        @pl.when(i > 0)
        def _():
            scatter_wait(msg1, colb1, sem_s1)

        compute(gath1, bond1, msg1)
        save_cols(idx1, colb1)
        scatter(msg1, colb1, sem_s1)

        @pl.when(i < NBLK // 2 - 1)
        def _():
            pltpu.async_copy(packed_h.at[gb + 3], idx1, sem_i1)

    scatter_wait(msg0, colb0, sem_s0)
    scatter_wait(msg1, colb1, sem_s1)
    plsc.subcore_barrier()
    pltpu.sync_copy(agg_sh.at[pl.ds(s * ROWS_PT, ROWS_PT)],
                    aggp_h.at[c, pl.ds(s * ROWS_PT, ROWS_PT)])


def _msg_kernel(hw2cat, btab, packed, zeros):
    return pl.kernel(
        _msg_body,
        out_type=jax.ShapeDtypeStruct((NC, NPAD, D), jnp.float32),
        mesh=_MESH,
        scratch_types=[
            pltpu.VMEM_SHARED((NPAD, D), jnp.float32),
            pltpu.VMEM((3, KP), jnp.int32),
            pltpu.VMEM((3, KP), jnp.int32),
            pltpu.VMEM((K, 2 * D), jnp.float32),
            pltpu.VMEM((K, 2 * D), jnp.float32),
            pltpu.VMEM((K, D), jnp.float32),
            pltpu.VMEM((K, D), jnp.float32),
            pltpu.VMEM((K, D), jnp.float32),
            pltpu.VMEM((K, D), jnp.float32),
            pltpu.VMEM((KP,), jnp.int32),
            pltpu.VMEM((KP,), jnp.int32),
            pltpu.SemaphoreType.DMA,
            pltpu.SemaphoreType.DMA,
            pltpu.SemaphoreType.DMA,
            pltpu.SemaphoreType.DMA,
            pltpu.SemaphoreType.DMA,
            pltpu.SemaphoreType.DMA,
            pltpu.SemaphoreType.DMA,
            pltpu.SemaphoreType.DMA,
        ],
        compiler_params=_SC_PARAMS,
    )(hw2cat, btab, packed, zeros)


_NB = 1000
_NBLKS = N // _NB


def _tc_deg_body(degp_ref, discol_ref, deginv_ref):
    deg = _dot(degp_ref[:, :N], jnp.ones((NW, 1), jnp.float32),
               ((0,), (0,))) + 1.0
    discol_ref[...] = jnp.broadcast_to(lax.rsqrt(deg), (N, 16))
    deginv_ref[...] = jnp.broadcast_to(1.0 / deg, (N, 16))


def _tc_deg(degp):
    return pl.pallas_call(
        _tc_deg_body,
        out_shape=[
            jax.ShapeDtypeStruct((N, 16), jnp.float32),
            jax.ShapeDtypeStruct((N, 16), jnp.float32),
        ],
    )(degp)


def _tc_prep_body(x_ref, at_ref, discol_ref, deginv_ref, w0_ref, root0_ref,
                  hw2_ref, self0_ref):
    h0 = jnp.zeros((_NB, D), jnp.float32)
    for f in range(9):
        oh = (x_ref[:, f:f + 1]
              == lax.broadcasted_iota(jnp.int32, (1, 120), 1)).astype(jnp.float32)
        h0 = h0 + _dot(oh, at_ref[f], ((1,), (0,)))
    dis = discol_ref[:, 0:1]
    dinv = deginv_ref[:, 0:1]
    hw0 = _dot_t(h0, w0_ref[...])
    hw2_ref[:, :D] = hw0 * dis
    hw2_ref[:, D:] = jnp.broadcast_to(dis, (_NB, D))
    self0_ref[...] = jnp.maximum(hw0 + root0_ref[...], 0.0) * dinv


def _tc_prep(x, atom_table, discol, deginv, W0, root0):
    return pl.pallas_call(
        _tc_prep_body,
        grid=(_NBLKS,),
        in_specs=[
            pl.BlockSpec((_NB, 9), lambda i: (i, 0)),
            pl.BlockSpec((9, 120, D), lambda i: (0, 0, 0)),
            pl.BlockSpec((_NB, 16), lambda i: (i, 0)),
            pl.BlockSpec((_NB, 16), lambda i: (i, 0)),
            pl.BlockSpec((D, D), lambda i: (0, 0)),
            pl.BlockSpec((1, D), lambda i: (0, 0)),
        ],
        out_specs=[
            pl.BlockSpec((_NB, 2 * D), lambda i: (i, 0)),
            pl.BlockSpec((_NB, D), lambda i: (i, 0)),
        ],
        out_shape=[
            jax.ShapeDtypeStruct((N, 2 * D), jnp.float32),
            jax.ShapeDtypeStruct((N, D), jnp.float32),
        ],
    )(x, atom_table, discol, deginv, W0, root0)


def _h_block(aggp_ref, discol_ref, self_ref):
    return ((aggp_ref[0] + aggp_ref[1]) * discol_ref[:, 0:1] + self_ref[...])


def _tc_stats_body(aggp_ref, discol_ref, self_ref, sums_ref):
    i = pl.program_id(0)

    @pl.when(i == 0)
    def _():
        sums_ref[...] = jnp.zeros((2, D), jnp.float32)

    h = _h_block(aggp_ref, discol_ref, self_ref)
    sums_ref[0:1, :] += jnp.sum(h, axis=0, keepdims=True)
    sums_ref[1:2, :] += jnp.sum(h * h, axis=0, keepdims=True)


def _tc_stats(aggp, discol, selfh):
    return pl.pallas_call(
        _tc_stats_body,
        grid=(_NBLKS,),
        in_specs=[
            pl.BlockSpec((NC, _NB, D), lambda i: (0, i, 0)),
            pl.BlockSpec((_NB, 16), lambda i: (i, 0)),
            pl.BlockSpec((_NB, D), lambda i: (i, 0)),
        ],
        out_specs=pl.BlockSpec((2, D), lambda i: (0, 0)),
        out_shape=jax.ShapeDtypeStruct((2, D), jnp.float32),
    )(aggp, discol, selfh)


def _tc_apply_body(aggp_ref, discol_ref, deginv_ref, self_ref, sums_ref,
                   g_ref, b_ref, wn_ref, rootn_ref, hw2_ref, selfn_ref):
    mean = sums_ref[0:1, :] / N
    var = sums_ref[1:2, :] / N - mean * mean
    scale = lax.rsqrt(var + 1e-5) * g_ref[...]
    shift = b_ref[...] - mean * scale
    dis = discol_ref[:, 0:1]
    dinv = deginv_ref[:, 0:1]
    h = jnp.maximum(_h_block(aggp_ref, discol_ref, self_ref) * scale + shift,
                    0.0)
    hwn = _dot_t(h, wn_ref[...])
    hw2_ref[:, :D] = hwn * dis
    hw2_ref[:, D:] = jnp.broadcast_to(dis, (_NB, D))
    selfn_ref[...] = jnp.maximum(hwn + rootn_ref[...], 0.0) * dinv


def _tc_combine(aggp, discol, deginv, selfh, bn_g, bn_b, Wn, rootn):
    sums = _tc_stats(aggp, discol, selfh)
    return pl.pallas_call(
        _tc_apply_body,
        grid=(_NBLKS,),
        in_specs=[
            pl.BlockSpec((NC, _NB, D), lambda i: (0, i, 0)),
            pl.BlockSpec((_NB, 16), lambda i: (i, 0)),
            pl.BlockSpec((_NB, 16), lambda i: (i, 0)),
            pl.BlockSpec((_NB, D), lambda i: (i, 0)),
            pl.BlockSpec((2, D), lambda i: (0, 0)),
            pl.BlockSpec((1, D), lambda i: (0, 0)),
            pl.BlockSpec((1, D), lambda i: (0, 0)),
            pl.BlockSpec((D, D), lambda i: (0, 0)),
            pl.BlockSpec((1, D), lambda i: (0, 0)),
        ],
        out_specs=[
            pl.BlockSpec((_NB, 2 * D), lambda i: (i, 0)),
            pl.BlockSpec((_NB, D), lambda i: (i, 0)),
        ],
        out_shape=[
            jax.ShapeDtypeStruct((N, 2 * D), jnp.float32),
            jax.ShapeDtypeStruct((N, D), jnp.float32),
        ],
    )(aggp, discol, deginv, selfh, sums, bn_g, bn_b, Wn, rootn)


def _tc_final_body(aggp_ref, discol_ref, self_ref, batch_ref, wp_ref, out_ref,
                   pooled_sc, counts_sc):
    i = pl.program_id(0)

    @pl.when(i == 0)
    def _():
        pooled_sc[...] = jnp.zeros((G, D), jnp.float32)
        counts_sc[...] = jnp.zeros((G, 1), jnp.float32)

    h = _h_block(aggp_ref, discol_ref, self_ref)
    oh = (batch_ref[...]
          == lax.broadcasted_iota(jnp.int32, (1, G), 1)).astype(jnp.float32)
    pooled_sc[...] += _dot(oh, h, ((0,), (0,)))
    counts_sc[...] += _dot(oh, jnp.ones((_NB, 1), jnp.float32), ((0,), (0,)))

    @pl.when(i == _NBLKS - 1)
    def _():
        pooled = pooled_sc[...] / jnp.maximum(counts_sc[:, 0:1], 1.0)
        out_ref[...] = _dot_t(pooled, wp_ref[...])


def _tc_final(aggp, discol, selfh, batch2, Wp):
    return pl.pallas_call(
        _tc_final_body,
        grid=(_NBLKS,),
        in_specs=[
            pl.BlockSpec((NC, _NB, D), lambda i: (0, i, 0)),
            pl.BlockSpec((_NB, 16), lambda i: (i, 0)),
            pl.BlockSpec((_NB, D), lambda i: (i, 0)),
            pl.BlockSpec((_NB, 1), lambda i: (i, 0)),
            pl.BlockSpec((D, D), lambda i: (0, 0)),
        ],
        out_specs=pl.BlockSpec((G, D), lambda i: (0, 0)),
        out_shape=jax.ShapeDtypeStruct((G, Wp.shape[0]), jnp.float32),
        scratch_shapes=[
            pltpu.VMEM((G, D), jnp.float32),
            pltpu.VMEM((G, 1), jnp.float32),
        ],
    )(aggp, discol, selfh, batch2, Wp)


def _btab(bond_table):
    t = (bond_table[0][:, None, None, :] + bond_table[1][None, :, None, :]
         + bond_table[2][None, None, :, :]).reshape(125, D)
    return jnp.pad(t, ((0, 3), (0, 0)))


def kernel(x, edge_index, edge_attr, batch, atom_table, W0, root0, bond0,
           W1, root1, bond1, W2, root2, bond2, bn0_g, bn0_b, bn1_g, bn1_b, Wp):
    row = edge_index[0].astype(jnp.int32)
    col = edge_index[1].astype(jnp.int32)
    combo = (edge_attr[:, 0] * 25 + edge_attr[:, 1] * 5
             + edge_attr[:, 2]).astype(jnp.int32)
    packed = jnp.stack(
        [row.reshape(NW, NBLK, K), col.reshape(NW, NBLK, K),
         combo.reshape(NW, NBLK, K)], axis=2).reshape(NW * NBLK, 3, K)
    packed = jnp.pad(packed, ((0, 0), (0, 0), (0, KP - K)))
    zeros = jnp.zeros((ROWS_PT, D), jnp.float32)
    zerosn = jnp.zeros((NPAD,), jnp.float32)

    degp = _deg_kernel(row, zerosn)
    discol, deginv = _tc_deg(degp)
    hw2, selfh = _tc_prep(x.astype(jnp.int32), atom_table, discol, deginv,
                          W0, root0)

    aggp = _msg_kernel(hw2, _btab(bond0), packed, zeros)
    hw2, selfh = _tc_combine(aggp, discol, deginv, selfh, bn0_g.reshape(1, D),
                             bn0_b.reshape(1, D), W1, root1)

    aggp = _msg_kernel(hw2, _btab(bond1), packed, zeros)
    hw2, selfh = _tc_combine(aggp, discol, deginv, selfh, bn1_g.reshape(1, D),
                             bn1_b.reshape(1, D), W2, root2)

    aggp = _msg_kernel(hw2, _btab(bond2), packed, zeros)
    return _tc_final(aggp, discol, selfh, batch.astype(jnp.int32).reshape(N, 1),
                     Wp)
